# Initial kernel scaffold; baseline (speedup 1.0000x reference)
import jax
import jax.numpy as jnp
from jax.experimental import pallas as pl


def kernel(hidden_states, top_k_index, top_k_weights, gate_w, up_w, down_w):
    raise NotImplementedError("write your pallas kernel here")


# speedup vs baseline: 3.1616x; 3.1616x over previous
import functools

import jax
import jax.numpy as jnp
from jax import lax
from jax.experimental import pallas as pl
from jax.experimental.pallas import tpu as pltpu
from jax.experimental.pallas import tpu_sc as plsc

T = 2048
D = 1024
F = 768
E = 64
CAP = 128
S = E * CAP

NC = 2
NS = 16
NW = NC * NS


def _gather_rows_kernel(n_rows, n_cols, chunk):
    per_w = n_rows // NW
    n_ch = per_w // chunk
    mesh = plsc.VectorSubcoreMesh(core_axis_name="c", subcore_axis_name="s")

    @functools.partial(
        pl.kernel,
        out_type=jax.ShapeDtypeStruct((n_rows, n_cols), jnp.float32),
        mesh=mesh,
        scratch_types=[
            pltpu.VMEM((per_w,), jnp.int32),
            pltpu.VMEM((chunk, n_cols), jnp.float32),
            pltpu.SemaphoreType.DMA,
        ],
    )
    def gather_k(table_hbm, idx_hbm, out_hbm, idx_v, rows_v, sem):
        wid = lax.axis_index("s") * NC + lax.axis_index("c")
        base = wid * per_w
        pltpu.sync_copy(idx_hbm.at[pl.ds(base, per_w)], idx_v)

        def body(i, carry):
            off = i * chunk
            pltpu.async_copy(table_hbm.at[idx_v.at[pl.ds(off, chunk)]], rows_v,
                             sem).wait()
            pltpu.sync_copy(rows_v, out_hbm.at[pl.ds(base + off, chunk)])
            return carry

        lax.fori_loop(0, n_ch, body, 0)

    return gather_k


@functools.cache
def _dispatch_gather():
    return _gather_rows_kernel(S, D, 64)


@functools.cache
def _combine_gather():
    return _gather_rows_kernel(T, D, 64)


def _mlp_body(x_ref, g_ref, u_ref, d_ref, w_ref, o_ref):
    e = pl.program_id(0)

    @pl.when(e == E)
    def _zero():
        o_ref[...] = jnp.zeros_like(o_ref)

    @pl.when(e < E)
    def _compute():
        x = x_ref[0]
        g = lax.dot_general(x, g_ref[0], (((1,), (1,)), ((), ())),
                            preferred_element_type=jnp.float32)
        u = lax.dot_general(x, u_ref[0], (((1,), (1,)), ((), ())),
                            preferred_element_type=jnp.float32)
        a = (g * jax.nn.sigmoid(g)) * u
        h = lax.dot_general(a, d_ref[0], (((1,), (1,)), ((), ())),
                            preferred_element_type=jnp.float32)
        o_ref[0] = h * w_ref[0, 0][:, None]


_mlp_call = pl.pallas_call(
    _mlp_body,
    grid=(E + 1,),
    in_specs=[
        pl.BlockSpec((1, CAP, D), lambda e: (jnp.minimum(e, E - 1), 0, 0)),
        pl.BlockSpec((1, F, D), lambda e: (jnp.minimum(e, E - 1), 0, 0)),
        pl.BlockSpec((1, F, D), lambda e: (jnp.minimum(e, E - 1), 0, 0)),
        pl.BlockSpec((1, D, F), lambda e: (jnp.minimum(e, E - 1), 0, 0)),
        pl.BlockSpec((1, 1, CAP), lambda e: (jnp.minimum(e, E - 1), 0, 0)),
    ],
    out_specs=pl.BlockSpec((1, CAP, D), lambda e: (e, 0, 0)),
    out_shape=jax.ShapeDtypeStruct((E + 1, CAP, D), jnp.float32),
    compiler_params=pltpu.CompilerParams(
        dimension_semantics=("arbitrary",)),
)


def kernel(hidden_states, top_k_index, top_k_weights, gate_w, up_w, down_w):
    idx = top_k_index[:, 0].astype(jnp.int32)
    wts = top_k_weights[:, 0]

    oh = (idx[:, None] == jnp.arange(E, dtype=jnp.int32)[None, :])
    pos = jnp.cumsum(oh.astype(jnp.int32), axis=0) - 1
    p = jnp.take_along_axis(pos, idx[:, None], axis=1)[:, 0]
    keep = p < CAP
    slot = jnp.where(keep, idx * CAP + p, S)

    arange_t = jnp.arange(T, dtype=jnp.int32)
    tok = jnp.zeros((S + 1,), jnp.int32).at[slot].set(arange_t)[:S]
    w_all = jnp.zeros((S + 1,), jnp.float32).at[slot].set(wts)[:S]

    x_all = _dispatch_gather()(hidden_states, tok)
    y = _mlp_call(x_all.reshape(E, CAP, D), gate_w, up_w, down_w,
                  w_all.reshape(E, 1, CAP))
    out = _combine_gather()(y.reshape((E + 1) * CAP, D), slot)
    return out


# speedup vs baseline: 5.9287x; 1.8753x over previous
import functools

import jax
import jax.numpy as jnp
from jax import lax
from jax.experimental import pallas as pl
from jax.experimental.pallas import tpu as pltpu
from jax.experimental.pallas import tpu_sc as plsc

T = 2048
D = 1024
F = 768
E = 64
CAP = 128
S = E * CAP

NC = 2
NS = 16
NW = NC * NS


def _gather_rows_kernel(n_rows, n_cols, chunk):
    per_w = n_rows // NW
    n_ch = per_w // chunk
    mesh = plsc.VectorSubcoreMesh(core_axis_name="c", subcore_axis_name="s")

    @functools.partial(
        pl.kernel,
        out_type=jax.ShapeDtypeStruct((n_rows, n_cols), jnp.float32),
        mesh=mesh,
        scratch_types=[
            pltpu.VMEM((per_w,), jnp.int32),
            pltpu.VMEM((chunk, n_cols), jnp.float32),
            pltpu.SemaphoreType.DMA,
        ],
    )
    def gather_k(table_hbm, idx_hbm, out_hbm, idx_v, rows_v, sem):
        wid = lax.axis_index("s") * NC + lax.axis_index("c")
        base = wid * per_w
        pltpu.sync_copy(idx_hbm.at[pl.ds(base, per_w)], idx_v)

        def body(i, carry):
            off = i * chunk
            pltpu.async_copy(table_hbm.at[idx_v.at[pl.ds(off, chunk)]], rows_v,
                             sem).wait()
            pltpu.sync_copy(rows_v, out_hbm.at[pl.ds(base + off, chunk)])
            return carry

        lax.fori_loop(0, n_ch, body, 0)

    return gather_k


@functools.cache
def _dispatch_gather():
    return _gather_rows_kernel(S, D, 64)


@functools.cache
def _combine_gather():
    return _gather_rows_kernel(T, D, 64)


def _mlp_body(x_ref, g_ref, u_ref, d_ref, w_ref, o_ref):
    e = pl.program_id(0)

    @pl.when(e == E)
    def _zero():
        o_ref[...] = jnp.zeros_like(o_ref)

    @pl.when(e < E)
    def _compute():
        x = x_ref[0]
        g = lax.dot_general(x, g_ref[0], (((1,), (1,)), ((), ())),
                            preferred_element_type=jnp.float32)
        u = lax.dot_general(x, u_ref[0], (((1,), (1,)), ((), ())),
                            preferred_element_type=jnp.float32)
        a = (g * jax.nn.sigmoid(g)) * u
        h = lax.dot_general(a, d_ref[0], (((1,), (1,)), ((), ())),
                            preferred_element_type=jnp.float32)
        o_ref[0] = h * w_ref[0, 0][:, None]


_mlp_call = pl.pallas_call(
    _mlp_body,
    grid=(E + 1,),
    in_specs=[
        pl.BlockSpec((1, CAP, D), lambda e: (jnp.minimum(e, E - 1), 0, 0)),
        pl.BlockSpec((1, F, D), lambda e: (jnp.minimum(e, E - 1), 0, 0)),
        pl.BlockSpec((1, F, D), lambda e: (jnp.minimum(e, E - 1), 0, 0)),
        pl.BlockSpec((1, D, F), lambda e: (jnp.minimum(e, E - 1), 0, 0)),
        pl.BlockSpec((1, 1, CAP), lambda e: (jnp.minimum(e, E - 1), 0, 0)),
    ],
    out_specs=pl.BlockSpec((1, CAP, D), lambda e: (e, 0, 0)),
    out_shape=jax.ShapeDtypeStruct((E + 1, CAP, D), jnp.float32),
    compiler_params=pltpu.CompilerParams(
        dimension_semantics=("arbitrary",)),
)


def kernel(hidden_states, top_k_index, top_k_weights, gate_w, up_w, down_w):
    idx = top_k_index[:, 0].astype(jnp.int32)
    wts = top_k_weights[:, 0]

    oh = (idx[:, None] == jnp.arange(E, dtype=jnp.int32)[None, :])
    pos = jnp.cumsum(oh.astype(jnp.int32), axis=0) - 1
    p = jnp.take_along_axis(pos, idx[:, None], axis=1)[:, 0]
    keep = p < CAP
    slot = jnp.where(keep, idx * CAP + p, S)

    arange_t = jnp.arange(T, dtype=jnp.int32)
    fill = jnp.arange(S + 1, dtype=jnp.int32) % T
    tok = fill.at[slot].set(arange_t)[:S]
    w_all = jnp.zeros((S + 1,), jnp.float32).at[slot].set(wts)[:S]

    x_all = _dispatch_gather()(hidden_states, tok)
    y = _mlp_call(x_all.reshape(E, CAP, D), gate_w, up_w, down_w,
                  w_all.reshape(E, 1, CAP))
    out = _combine_gather()(y.reshape((E + 1) * CAP, D), slot)
    return out


# speedup vs baseline: 5.9568x; 1.0047x over previous
import functools

import jax
import jax.numpy as jnp
from jax import lax
from jax.experimental import pallas as pl
from jax.experimental.pallas import tpu as pltpu
from jax.experimental.pallas import tpu_sc as plsc

T = 2048
D = 1024
F = 768
E = 64
CAP = 128
S = E * CAP

NC = 2
NS = 16
NW = NC * NS


def _gather_rows_kernel(n_rows, n_cols, chunk):
    per_w = n_rows // NW
    n_ch = per_w // chunk
    mesh = plsc.VectorSubcoreMesh(core_axis_name="c", subcore_axis_name="s")

    @functools.partial(
        pl.kernel,
        out_type=jax.ShapeDtypeStruct((n_rows, n_cols), jnp.float32),
        mesh=mesh,
        scratch_types=[
            pltpu.VMEM((per_w,), jnp.int32),
            pltpu.VMEM((chunk, n_cols), jnp.float32),
            pltpu.SemaphoreType.DMA,
        ],
    )
    def gather_k(table_hbm, idx_hbm, out_hbm, idx_v, rows_v, sem):
        wid = lax.axis_index("s") * NC + lax.axis_index("c")
        base = wid * per_w
        pltpu.sync_copy(idx_hbm.at[pl.ds(base, per_w)], idx_v)

        def body(i, carry):
            off = i * chunk
            pltpu.async_copy(table_hbm.at[idx_v.at[pl.ds(off, chunk)]], rows_v,
                             sem).wait()
            pltpu.sync_copy(rows_v, out_hbm.at[pl.ds(base + off, chunk)])
            return carry

        lax.fori_loop(0, n_ch, body, 0)

    return gather_k


@functools.cache
def _dispatch_gather():
    return _gather_rows_kernel(S, D, 64)


@functools.cache
def _combine_gather():
    return _gather_rows_kernel(T, D, 64)


EPB = 2
NSTEP = E // EPB + 1


def _mlp_body(x_ref, g_ref, u_ref, d_ref, w_ref, o_ref):
    e = pl.program_id(0)

    @pl.when(e == NSTEP - 1)
    def _zero():
        o_ref[...] = jnp.zeros_like(o_ref)

    @pl.when(e < NSTEP - 1)
    def _compute():
        for j in range(EPB):
            x = x_ref[j]
            g = lax.dot_general(x, g_ref[j], (((1,), (1,)), ((), ())),
                                preferred_element_type=jnp.float32)
            u = lax.dot_general(x, u_ref[j], (((1,), (1,)), ((), ())),
                                preferred_element_type=jnp.float32)
            a = (g * jax.nn.sigmoid(g)) * u
            h = lax.dot_general(a, d_ref[j], (((1,), (1,)), ((), ())),
                                preferred_element_type=jnp.float32)
            o_ref[j] = h * w_ref[j, 0][:, None]


def _wmap(e):
    return (jnp.minimum(e, E // EPB - 1), 0, 0)


_mlp_call = pl.pallas_call(
    _mlp_body,
    grid=(NSTEP,),
    in_specs=[
        pl.BlockSpec((EPB, CAP, D), _wmap),
        pl.BlockSpec((EPB, F, D), _wmap),
        pl.BlockSpec((EPB, F, D), _wmap),
        pl.BlockSpec((EPB, D, F), _wmap),
        pl.BlockSpec((EPB, 1, CAP), _wmap),
    ],
    out_specs=pl.BlockSpec((EPB, CAP, D), lambda e: (e, 0, 0)),
    out_shape=jax.ShapeDtypeStruct((NSTEP * EPB, CAP, D), jnp.float32),
    compiler_params=pltpu.CompilerParams(
        dimension_semantics=("arbitrary",)),
)


def kernel(hidden_states, top_k_index, top_k_weights, gate_w, up_w, down_w):
    idx = top_k_index[:, 0].astype(jnp.int32)
    wts = top_k_weights[:, 0]

    oh = (idx[:, None] == jnp.arange(E, dtype=jnp.int32)[None, :])
    pos = jnp.cumsum(oh.astype(jnp.int32), axis=0) - 1
    p = jnp.take_along_axis(pos, idx[:, None], axis=1)[:, 0]
    keep = p < CAP
    slot = jnp.where(keep, idx * CAP + p, S)

    arange_t = jnp.arange(T, dtype=jnp.int32)
    fill = jnp.arange(S + 1, dtype=jnp.int32) % T
    tok = fill.at[slot].set(arange_t)[:S]
    w_all = jnp.zeros((S + 1,), jnp.float32).at[slot].set(wts)[:S]

    x_all = _dispatch_gather()(hidden_states, tok)
    y = _mlp_call(x_all.reshape(E, CAP, D), gate_w, up_w, down_w,
                  w_all.reshape(E, 1, CAP))
    out = _combine_gather()(y.reshape(NSTEP * EPB * CAP, D), slot)
    return out


# speedup vs baseline: 6.0342x; 1.0130x over previous
import functools

import jax
import jax.numpy as jnp
from jax import lax
from jax.experimental import pallas as pl
from jax.experimental.pallas import tpu as pltpu
from jax.experimental.pallas import tpu_sc as plsc

T = 2048
D = 1024
F = 768
E = 64
CAP = 128
S = E * CAP

NC = 2
NS = 16
NW = NC * NS


def _gather_rows_kernel(n_rows, n_cols, chunk):
    per_w = n_rows // NW
    n_ch = per_w // chunk
    mesh = plsc.VectorSubcoreMesh(core_axis_name="c", subcore_axis_name="s")

    @functools.partial(
        pl.kernel,
        out_type=jax.ShapeDtypeStruct((n_rows, n_cols), jnp.float32),
        mesh=mesh,
        scratch_types=[
            pltpu.VMEM((per_w,), jnp.int32),
            pltpu.VMEM((chunk, n_cols), jnp.float32),
            pltpu.VMEM((chunk, n_cols), jnp.float32),
            pltpu.SemaphoreType.DMA,
            pltpu.SemaphoreType.DMA,
        ],
    )
    def gather_k(table_hbm, idx_hbm, out_hbm, idx_v, rows_a, rows_b, sem_a,
                 sem_b):
        wid = lax.axis_index("s") * NC + lax.axis_index("c")
        base = wid * per_w
        pltpu.sync_copy(idx_hbm.at[pl.ds(base, per_w)], idx_v)
        bufs = (rows_a, rows_b)
        sems = (sem_a, sem_b)

        def start(i):
            return pltpu.async_copy(
                table_hbm.at[idx_v.at[pl.ds(i * chunk, chunk)]],
                bufs[i % 2], sems[i % 2])

        cps = [start(0)]
        for i in range(n_ch):
            if i + 1 < n_ch:
                cps.append(start(i + 1))
            cps[i].wait()
            pltpu.sync_copy(bufs[i % 2],
                            out_hbm.at[pl.ds(base + i * chunk, chunk)])

    return gather_k


@functools.cache
def _dispatch_gather():
    return _gather_rows_kernel(S, D, 32)


@functools.cache
def _combine_gather():
    return _gather_rows_kernel(T, D, 32)


EPB = 2
NSTEP = E // EPB + 1


def _mlp_body(x_ref, g_ref, u_ref, d_ref, w_ref, o_ref):
    e = pl.program_id(0)

    @pl.when(e == NSTEP - 1)
    def _zero():
        o_ref[...] = jnp.zeros_like(o_ref)

    @pl.when(e < NSTEP - 1)
    def _compute():
        for j in range(EPB):
            x = x_ref[j]
            g = lax.dot_general(x, g_ref[j], (((1,), (1,)), ((), ())),
                                preferred_element_type=jnp.float32)
            u = lax.dot_general(x, u_ref[j], (((1,), (1,)), ((), ())),
                                preferred_element_type=jnp.float32)
            a = (g * jax.nn.sigmoid(g)) * u
            h = lax.dot_general(a, d_ref[j], (((1,), (1,)), ((), ())),
                                preferred_element_type=jnp.float32)
            o_ref[j] = h * w_ref[j, 0][:, None]


def _wmap(e):
    return (jnp.minimum(e, E // EPB - 1), 0, 0)


_mlp_call = pl.pallas_call(
    _mlp_body,
    grid=(NSTEP,),
    in_specs=[
        pl.BlockSpec((EPB, CAP, D), _wmap),
        pl.BlockSpec((EPB, F, D), _wmap),
        pl.BlockSpec((EPB, F, D), _wmap),
        pl.BlockSpec((EPB, D, F), _wmap),
        pl.BlockSpec((EPB, 1, CAP), _wmap),
    ],
    out_specs=pl.BlockSpec((EPB, CAP, D), lambda e: (e, 0, 0)),
    out_shape=jax.ShapeDtypeStruct((NSTEP * EPB, CAP, D), jnp.float32),
    compiler_params=pltpu.CompilerParams(
        dimension_semantics=("arbitrary",)),
)


def kernel(hidden_states, top_k_index, top_k_weights, gate_w, up_w, down_w):
    idx = top_k_index[:, 0].astype(jnp.int32)
    wts = top_k_weights[:, 0]

    oh = (idx[:, None] == jnp.arange(E, dtype=jnp.int32)[None, :])
    pos = jnp.cumsum(oh.astype(jnp.int32), axis=0) - 1
    p = jnp.take_along_axis(pos, idx[:, None], axis=1)[:, 0]
    keep = p < CAP
    slot = jnp.where(keep, idx * CAP + p, S)

    arange_t = jnp.arange(T, dtype=jnp.int32)
    fill = jnp.arange(S + 1, dtype=jnp.int32) % T
    tok = fill.at[slot].set(arange_t)[:S]
    w_all = jnp.zeros((S + 1,), jnp.float32).at[slot].set(wts)[:S]

    x_all = _dispatch_gather()(hidden_states, tok)
    y = _mlp_call(x_all.reshape(E, CAP, D), gate_w, up_w, down_w,
                  w_all.reshape(E, 1, CAP))
    out = _combine_gather()(y.reshape(NSTEP * EPB * CAP, D), slot)
    return out


# speedup vs baseline: 6.5034x; 1.0778x over previous
import functools

import jax
import jax.numpy as jnp
from jax import lax
from jax.experimental import pallas as pl
from jax.experimental.pallas import tpu as pltpu
from jax.experimental.pallas import tpu_sc as plsc

T = 2048
D = 1024
F = 768
E = 64
CAP = 128
S = E * CAP

NC = 2
NS = 16
NW = NC * NS


def _gather_rows_kernel(n_rows, n_cols, chunk):
    per_w = n_rows // NW
    n_ch = per_w // chunk
    mesh = plsc.VectorSubcoreMesh(core_axis_name="c", subcore_axis_name="s")

    @functools.partial(
        pl.kernel,
        out_type=jax.ShapeDtypeStruct((n_rows, n_cols), jnp.float32),
        mesh=mesh,
        scratch_types=[
            pltpu.VMEM((per_w,), jnp.int32),
            pltpu.VMEM((chunk, n_cols), jnp.float32),
            pltpu.VMEM((chunk, n_cols), jnp.float32),
            pltpu.SemaphoreType.DMA,
            pltpu.SemaphoreType.DMA,
        ],
    )
    def gather_k(table_hbm, idx_hbm, out_hbm, idx_v, rows_a, rows_b, sem_a,
                 sem_b):
        wid = lax.axis_index("s") * NC + lax.axis_index("c")
        base = wid * per_w
        pltpu.sync_copy(idx_hbm.at[pl.ds(base, per_w)], idx_v)
        bufs = (rows_a, rows_b)
        sems = (sem_a, sem_b)

        def start(i):
            return pltpu.async_copy(
                table_hbm.at[idx_v.at[pl.ds(i * chunk, chunk)]],
                bufs[i % 2], sems[i % 2])

        cps = [start(0)]
        for i in range(n_ch):
            if i + 1 < n_ch:
                cps.append(start(i + 1))
            cps[i].wait()
            pltpu.sync_copy(bufs[i % 2],
                            out_hbm.at[pl.ds(base + i * chunk, chunk)])

    return gather_k


@functools.cache
def _dispatch_gather():
    return _gather_rows_kernel(S, D, 32)


@functools.cache
def _combine_gather():
    return _gather_rows_kernel(T, D, 32)


EPB = 2
NSTEP = E // EPB + 1


def _mlp_body(x_ref, g_ref, u_ref, d_ref, w_ref, o_ref):
    e = pl.program_id(0)

    @pl.when(e == NSTEP - 1)
    def _zero():
        o_ref[...] = jnp.zeros_like(o_ref)

    @pl.when(e < NSTEP - 1)
    def _compute():
        for j in range(EPB):
            x = x_ref[j]
            g = lax.dot_general(x, g_ref[j], (((1,), (1,)), ((), ())),
                                preferred_element_type=jnp.float32)
            u = lax.dot_general(x, u_ref[j], (((1,), (1,)), ((), ())),
                                preferred_element_type=jnp.float32)
            a = (g * jax.nn.sigmoid(g)) * u
            h = lax.dot_general(a, d_ref[j], (((1,), (1,)), ((), ())),
                                preferred_element_type=jnp.float32)
            o_ref[j] = h * w_ref[j, 0][:, None]


def _wmap(e):
    return (jnp.minimum(e, E // EPB - 1), 0, 0)


_mlp_call = pl.pallas_call(
    _mlp_body,
    grid=(NSTEP,),
    in_specs=[
        pl.BlockSpec((EPB, CAP, D), _wmap),
        pl.BlockSpec((EPB, F, D), _wmap),
        pl.BlockSpec((EPB, F, D), _wmap),
        pl.BlockSpec((EPB, D, F), _wmap),
        pl.BlockSpec((EPB, 1, CAP), _wmap),
    ],
    out_specs=pl.BlockSpec((EPB, CAP, D), lambda e: (e, 0, 0)),
    out_shape=jax.ShapeDtypeStruct((NSTEP * EPB, CAP, D), jnp.float32),
    compiler_params=pltpu.CompilerParams(
        dimension_semantics=("arbitrary",)),
)


def kernel(hidden_states, top_k_index, top_k_weights, gate_w, up_w, down_w):
    idx = top_k_index[:, 0].astype(jnp.int32)
    wts = top_k_weights[:, 0]

    G = 16
    GS = T // G
    oh = (idx[:, None] == jnp.arange(E, dtype=jnp.int32)[None, :])
    ohf = oh.astype(jnp.float32)
    ohg = ohf.reshape(G, GS, E)
    r = jnp.arange(GS, dtype=jnp.int32)
    tri = (r[:, None] >= r[None, :]).astype(jnp.float32)
    within = jnp.einsum('ij,gje->gie', tri, ohg,
                        preferred_element_type=jnp.float32)
    gsum = within[:, -1, :]
    offs = jnp.cumsum(gsum, axis=0) - gsum
    pos = (within + offs[:, None, :]).reshape(T, E)
    p = (jnp.sum(pos * ohf, axis=1) - 1.0).astype(jnp.int32)
    keep = p < CAP
    slot = jnp.where(keep, idx * CAP + p, S)

    arange_t = jnp.arange(T, dtype=jnp.int32)
    fill = jnp.arange(S + 1, dtype=jnp.int32) % T
    tok = fill.at[slot].set(arange_t)[:S]
    w_all = jnp.zeros((S + 1,), jnp.float32).at[slot].set(wts)[:S]

    x_all = _dispatch_gather()(hidden_states, tok)
    y = _mlp_call(x_all.reshape(E, CAP, D), gate_w, up_w, down_w,
                  w_all.reshape(E, 1, CAP))
    out = _combine_gather()(y.reshape(NSTEP * EPB * CAP, D), slot)
    return out
